# Initial kernel scaffold; baseline (speedup 1.0000x reference)
import jax
import jax.numpy as jnp
from jax.experimental import pallas as pl


def kernel(X, Q, top_k, W1, b1, W2, b2, W3, b3, D1, bd1, D2, bd2):
    raise NotImplementedError("write your pallas kernel here")


# speedup vs baseline: 2.9764x; 2.9764x over previous
import functools

import jax
import jax.numpy as jnp
from jax import lax
from jax.experimental import pallas as pl
from jax.experimental.pallas import tpu as pltpu
from jax.experimental.pallas import tpu_sc as plsc

N = 4096
NET = 4096
HID = 64
K = 20

NC = 2
NS = 16
NW = NC * NS
L = 16

RPW = N // NW
SUB = 32
NSUB = RPW // SUB
KSUB = SUB * K
IDX_CHUNK = 128
NIC = KSUB // IDX_CHUNK
ND = HID // L


def _enc_body(x_ref, w1_ref, b1_ref, w2_ref, b2_ref, w3_ref, b3_ref, h_ref):
    h1 = jnp.maximum(
        jnp.dot(x_ref[...], w1_ref[...], preferred_element_type=jnp.float32)
        + b1_ref[...], 0.0)
    h2 = jnp.maximum(
        jnp.dot(h1, w2_ref[...], preferred_element_type=jnp.float32)
        + b2_ref[...], 0.0)
    h_ref[...] = jnp.maximum(
        jnp.dot(h2, w3_ref[...], preferred_element_type=jnp.float32)
        + b3_ref[...], 0.0)


def _encoder(X, W1, b1, W2, b2, W3, b3):
    BM = 512
    return pl.pallas_call(
        _enc_body,
        grid=(N // BM,),
        in_specs=[
            pl.BlockSpec((BM, NET), lambda i: (i, 0)),
            pl.BlockSpec((NET, 256), lambda i: (0, 0)),
            pl.BlockSpec((1, 256), lambda i: (0, 0)),
            pl.BlockSpec((256, 84), lambda i: (0, 0)),
            pl.BlockSpec((1, 84), lambda i: (0, 0)),
            pl.BlockSpec((84, HID), lambda i: (0, 0)),
            pl.BlockSpec((1, HID), lambda i: (0, 0)),
        ],
        out_specs=pl.BlockSpec((BM, HID), lambda i: (i, 0)),
        out_shape=jax.ShapeDtypeStruct((N, HID), jnp.float32),
        compiler_params=pltpu.CompilerParams(
            dimension_semantics=("arbitrary",)),
    )(X, W1, b1.reshape(1, -1), W2, b2.reshape(1, -1), W3, b3.reshape(1, -1))


def _dec_body(z_ref, d1_ref, bd1_ref, d2_ref, bd2_ref, o_ref):
    hd = jnp.maximum(
        jnp.dot(z_ref[...], d1_ref[...], preferred_element_type=jnp.float32)
        + bd1_ref[...], 0.0)
    o_ref[...] = jnp.maximum(
        jnp.dot(hd, d2_ref[...], preferred_element_type=jnp.float32)
        + bd2_ref[...], 0.0)


def _decoder(Z, D1, bd1, D2, bd2):
    BM = 512
    return pl.pallas_call(
        _dec_body,
        grid=(N // BM,),
        in_specs=[
            pl.BlockSpec((BM, HID), lambda i: (i, 0)),
            pl.BlockSpec((HID, 256), lambda i: (0, 0)),
            pl.BlockSpec((1, 256), lambda i: (0, 0)),
            pl.BlockSpec((256, NET), lambda i: (0, 0)),
            pl.BlockSpec((1, NET), lambda i: (0, 0)),
        ],
        out_specs=pl.BlockSpec((BM, NET), lambda i: (i, 0)),
        out_shape=jax.ShapeDtypeStruct((N, NET), jnp.float32),
        compiler_params=pltpu.CompilerParams(
            dimension_semantics=("arbitrary",)),
    )(Z, D1, bd1.reshape(1, -1), D2, bd2.reshape(1, -1))


@functools.lru_cache(maxsize=None)
def _mesh():
    return plsc.VectorSubcoreMesh(
        core_axis_name="c", subcore_axis_name="s",
        num_cores=NC, num_subcores=NS)


_QSTAR_SCRATCH = [
    pltpu.VMEM((RPW * K,), jnp.int32),
    pltpu.VMEM((RPW * K,), jnp.int32),
    pltpu.VMEM((RPW * K,), jnp.int32),
    pltpu.VMEM((RPW, HID), jnp.float32),
    pltpu.VMEM((KSUB, HID), jnp.float32),
    pltpu.VMEM((KSUB,), jnp.float32),
    pltpu.VMEM((SUB, HID), jnp.float32),
    pltpu.SemaphoreType.DMA,
]


def _qstar_body(h_hbm, qflat_hbm, topk_hbm, rowid_hbm, z_hbm,
                topk_v, rowid_v, qidx_v, hown_v, rows_v, qw_v, z_v, sem):
    wid = lax.axis_index("s") * NC + lax.axis_index("c")
    base = wid * RPW
    kbase = base * K
    pltpu.sync_copy(topk_hbm.at[pl.ds(kbase, RPW * K)], topk_v)
    pltpu.sync_copy(rowid_hbm.at[pl.ds(kbase, RPW * K)], rowid_v)
    pltpu.sync_copy(h_hbm.at[pl.ds(base, RPW)], hown_v)

    def qidx_step(s, carry):
        tk = topk_v[pl.ds(s * L, L)]
        rid = rowid_v[pl.ds(s * L, L)]
        qidx_v[pl.ds(s * L, L)] = rid * N + tk
        return carry
    lax.fori_loop(0, RPW * K // L, qidx_step, 0)

    for c in range(NSUB):
        cb = c * KSUB
        cpys = []
        for j in range(NIC):
            cpys.append(pltpu.async_copy(
                h_hbm.at[topk_v.at[pl.ds(cb + j * IDX_CHUNK, IDX_CHUNK)]],
                rows_v.at[pl.ds(j * IDX_CHUNK, IDX_CHUNK)], sem))
            cpys.append(pltpu.async_copy(
                qflat_hbm.at[qidx_v.at[pl.ds(cb + j * IDX_CHUNK, IDX_CHUNK)]],
                qw_v.at[pl.ds(j * IDX_CHUNK, IDX_CHUNK)], sem))
        for cp in cpys:
            cp.wait()

        def node_step(n, carry):
            node = c * SUB + n
            kb = n * K
            accs = [hown_v[node, pl.ds(d * L, L)] for d in range(ND)]
            for k in range(K):
                b = plsc.load_gather(
                    qw_v, [jnp.full((L,), kb + k, jnp.int32)])
                for d in range(ND):
                    accs[d] = accs[d] + b * rows_v[kb + k, pl.ds(d * L, L)]
            for d in range(ND):
                z_v[n, pl.ds(d * L, L)] = accs[d]
            return carry
        lax.fori_loop(0, SUB, node_step, 0)
        pltpu.sync_copy(z_v, z_hbm.at[pl.ds(base + c * SUB, SUB)])


@functools.lru_cache(maxsize=None)
def _qstar():
    return pl.kernel(
        _qstar_body,
        out_type=jax.ShapeDtypeStruct((N, HID), jnp.float32),
        mesh=_mesh(),
        scratch_types=_QSTAR_SCRATCH,
        compiler_params=pltpu.CompilerParams(
            needs_layout_passes=False, use_tc_tiling_on_sc=False),
    )


def kernel(X, Q, top_k, W1, b1, W2, b2, W3, b3, D1, bd1, D2, bd2):
    H = _encoder(X, W1, b1, W2, b2, W3, b3)
    rowid = jnp.repeat(jnp.arange(N, dtype=jnp.int32), K)
    Z = _qstar()(H, Q.reshape(-1), top_k.reshape(-1), rowid)
    X_rec = _decoder(Z, D1, bd1, D2, bd2)
    return (X_rec, Z)


# speedup vs baseline: 3.8191x; 1.2831x over previous
import functools

import jax
import jax.numpy as jnp
from jax import lax
from jax.experimental import pallas as pl
from jax.experimental.pallas import tpu as pltpu
from jax.experimental.pallas import tpu_sc as plsc

N = 4096
NET = 4096
HID = 64
K = 20

NC = 2
NS = 16
NW = NC * NS
L = 16

RPW = N // NW
KPW = RPW * K

QG = 8
NQG = RPW // QG

SUB = 32
NSUB = RPW // SUB
KSUB = SUB * K
IDX_CHUNK = 128
NIC = KSUB // IDX_CHUNK
ND = HID // L


def _enc_body(x_ref, w1_ref, b1_ref, w2_ref, b2_ref, w3_ref, b3_ref, h_ref):
    h1 = jnp.maximum(
        jnp.dot(x_ref[...], w1_ref[...], preferred_element_type=jnp.float32)
        + b1_ref[...], 0.0)
    h2 = jnp.maximum(
        jnp.dot(h1, w2_ref[...], preferred_element_type=jnp.float32)
        + b2_ref[...], 0.0)
    h_ref[...] = jnp.maximum(
        jnp.dot(h2, w3_ref[...], preferred_element_type=jnp.float32)
        + b3_ref[...], 0.0)


def _encoder(X, W1, b1, W2, b2, W3, b3):
    BM = 512
    return pl.pallas_call(
        _enc_body,
        grid=(N // BM,),
        in_specs=[
            pl.BlockSpec((BM, NET), lambda i: (i, 0)),
            pl.BlockSpec((NET, 256), lambda i: (0, 0)),
            pl.BlockSpec((1, 256), lambda i: (0, 0)),
            pl.BlockSpec((256, 84), lambda i: (0, 0)),
            pl.BlockSpec((1, 84), lambda i: (0, 0)),
            pl.BlockSpec((84, HID), lambda i: (0, 0)),
            pl.BlockSpec((1, HID), lambda i: (0, 0)),
        ],
        out_specs=pl.BlockSpec((BM, HID), lambda i: (i, 0)),
        out_shape=jax.ShapeDtypeStruct((N, HID), jnp.float32),
        compiler_params=pltpu.CompilerParams(
            dimension_semantics=("arbitrary",)),
    )(X, W1, b1.reshape(1, -1), W2, b2.reshape(1, -1), W3, b3.reshape(1, -1))


def _dec_body(z_ref, d1_ref, bd1_ref, d2_ref, bd2_ref, o_ref):
    hd = jnp.maximum(
        jnp.dot(z_ref[...], d1_ref[...], preferred_element_type=jnp.float32)
        + bd1_ref[...], 0.0)
    o_ref[...] = jnp.maximum(
        jnp.dot(hd, d2_ref[...], preferred_element_type=jnp.float32)
        + bd2_ref[...], 0.0)


def _decoder(Z, D1, bd1, D2, bd2):
    BM = 512
    return pl.pallas_call(
        _dec_body,
        grid=(N // BM,),
        in_specs=[
            pl.BlockSpec((BM, HID), lambda i: (i, 0)),
            pl.BlockSpec((HID, 256), lambda i: (0, 0)),
            pl.BlockSpec((1, 256), lambda i: (0, 0)),
            pl.BlockSpec((256, NET), lambda i: (0, 0)),
            pl.BlockSpec((1, NET), lambda i: (0, 0)),
        ],
        out_specs=pl.BlockSpec((BM, NET), lambda i: (i, 0)),
        out_shape=jax.ShapeDtypeStruct((N, NET), jnp.float32),
        compiler_params=pltpu.CompilerParams(
            dimension_semantics=("arbitrary",)),
    )(Z, D1, bd1.reshape(1, -1), D2, bd2.reshape(1, -1))


@functools.lru_cache(maxsize=None)
def _mesh():
    return plsc.VectorSubcoreMesh(
        core_axis_name="c", subcore_axis_name="s",
        num_cores=NC, num_subcores=NS)


_QW_SCRATCH = [
    pltpu.VMEM((KPW,), jnp.int32),
    pltpu.VMEM((KPW,), jnp.float32),
    pltpu.VMEM((2, QG, NET), jnp.float32),
    pltpu.SemaphoreType.DMA,
    pltpu.SemaphoreType.DMA,
]


def _qw_body(q_hbm, topk_hbm, qw_hbm, topk_v, qw_v, qr_v, sem0, sem1):
    wid = lax.axis_index("s") * NC + lax.axis_index("c")
    base = wid * RPW
    kbase = base * K
    pltpu.sync_copy(topk_hbm.at[pl.ds(kbase, KPW)], topk_v)
    sems = (sem0, sem1)

    def fire(g, s):
        def _enq():
            pltpu.async_copy(
                q_hbm.at[pl.ds(base + g * QG, QG)], qr_v.at[s], sems[s])
        if isinstance(g, int):
            if g < NQG:
                _enq()
        else:
            pl.when(g < NQG)(_enq)

    def drain(s):
        pltpu.make_async_copy(
            q_hbm.at[pl.ds(base, QG)], qr_v.at[s], sems[s]).wait()

    fire(0, 0)
    fire(1, 1)

    def group_pair(i, carry):
        for s in range(2):
            g = 2 * i + s
            drain(s)
            for n in range(QG):
                kb = g * QG * K + n * K
                tk0 = topk_v[pl.ds(kb, L)]
                tk1 = topk_v[pl.ds(kb + K - L, L)]
                row = jnp.full((L,), n, jnp.int32)
                g0 = plsc.load_gather(qr_v.at[s], [row, tk0])
                g1 = plsc.load_gather(qr_v.at[s], [row, tk1])
                qw_v[pl.ds(kb, L)] = g0
                qw_v[pl.ds(kb + K - L, L)] = g1
            fire(g + 2, s)
        return carry
    lax.fori_loop(0, NQG // 2, group_pair, 0)
    pltpu.sync_copy(qw_v, qw_hbm.at[pl.ds(kbase, KPW)])


@functools.lru_cache(maxsize=None)
def _qw_extract():
    return pl.kernel(
        _qw_body,
        out_type=jax.ShapeDtypeStruct((N * K,), jnp.float32),
        mesh=_mesh(),
        scratch_types=_QW_SCRATCH,
        compiler_params=pltpu.CompilerParams(needs_layout_passes=False),
    )


_AGG_SCRATCH = [
    pltpu.VMEM((KPW,), jnp.int32),
    pltpu.VMEM((KPW,), jnp.float32),
    pltpu.VMEM((RPW, HID), jnp.float32),
    pltpu.VMEM((2, KSUB, HID), jnp.float32),
    pltpu.VMEM((SUB, HID), jnp.float32),
    pltpu.SemaphoreType.DMA,
    pltpu.SemaphoreType.DMA,
]


def _agg_body(h_hbm, topk_hbm, qw_hbm, z_hbm,
              topk_v, qw_v, hown_v, rows_v, z_v, sem0, sem1):
    wid = lax.axis_index("s") * NC + lax.axis_index("c")
    base = wid * RPW
    kbase = base * K
    pltpu.sync_copy(topk_hbm.at[pl.ds(kbase, KPW)], topk_v)
    pltpu.sync_copy(qw_hbm.at[pl.ds(kbase, KPW)], qw_v)
    pltpu.sync_copy(h_hbm.at[pl.ds(base, RPW)], hown_v)
    sems = (sem0, sem1)

    def fire(c, s):
        if c >= NSUB:
            return
        cb = c * KSUB
        for j in range(NIC):
            pltpu.async_copy(
                h_hbm.at[topk_v.at[pl.ds(cb + j * IDX_CHUNK, IDX_CHUNK)]],
                rows_v.at[s, pl.ds(j * IDX_CHUNK, IDX_CHUNK)], sems[s])

    def drain(s):
        for j in range(NIC):
            pltpu.make_async_copy(
                h_hbm.at[pl.ds(0, IDX_CHUNK)],
                rows_v.at[s, pl.ds(j * IDX_CHUNK, IDX_CHUNK)], sems[s]).wait()

    fire(0, 0)
    for c in range(NSUB):
        s = c & 1
        drain(s)
        fire(c + 1, s ^ 1)

        def node_step(n, carry, s=s, c=c):
            node = c * SUB + n
            kb = n * K
            qa = qw_v[pl.ds(node * K, L)]
            qb = qw_v[pl.ds(node * K + K - L, L)]
            accs = [hown_v[node, pl.ds(d * L, L)] for d in range(ND)]
            for k in range(K):
                if k < L:
                    b = jnp.broadcast_to(qa[k], (L,))
                else:
                    b = jnp.broadcast_to(qb[k - (K - L)], (L,))
                for d in range(ND):
                    accs[d] = accs[d] + b * rows_v[s, kb + k, pl.ds(d * L, L)]
            for d in range(ND):
                z_v[n, pl.ds(d * L, L)] = accs[d]
            return carry
        lax.fori_loop(0, SUB, node_step, 0)
        pltpu.sync_copy(z_v, z_hbm.at[pl.ds(base + c * SUB, SUB)])


@functools.lru_cache(maxsize=None)
def _aggregate():
    return pl.kernel(
        _agg_body,
        out_type=jax.ShapeDtypeStruct((N, HID), jnp.float32),
        mesh=_mesh(),
        scratch_types=_AGG_SCRATCH,
        compiler_params=pltpu.CompilerParams(
            needs_layout_passes=False, use_tc_tiling_on_sc=False),
    )


def kernel(X, Q, top_k, W1, b1, W2, b2, W3, b3, D1, bd1, D2, bd2):
    H = _encoder(X, W1, b1, W2, b2, W3, b3)
    topk_flat = top_k.reshape(-1)
    qw = _qw_extract()(Q, topk_flat)
    Z = _aggregate()(H, topk_flat, qw)
    X_rec = _decoder(Z, D1, bd1, D2, bd2)
    return (X_rec, Z)
